# baseline (speedup 1.0000x reference)
import functools

import jax
import jax.numpy as jnp
from jax import lax
from jax.experimental import pallas as pl
from jax.experimental.pallas import tpu as pltpu
from jax.experimental.pallas import tpu_sc as plsc

_L = 16
_D = 4


@functools.lru_cache(maxsize=None)
def _make_sc_kernel(N, H, V, CH):
    info = plsc.get_sparse_core_info()
    NC, NS = info.num_cores, info.num_subcores
    NW = NC * NS
    assert N % (NW * CH) == 0 and H % _L == 0
    rows_per_w = N // NW
    n_chunks = rows_per_w // CH
    assert n_chunks % _D == 0 and n_chunks // _D >= 3
    quads = n_chunks // _D
    groups = H // _L
    CHH = CH * H
    mesh = plsc.VectorSubcoreMesh(core_axis_name="c", subcore_axis_name="s")

    @functools.partial(
        pl.kernel,
        mesh=mesh,
        out_type=jax.ShapeDtypeStruct((N, H), jnp.float32),
        compiler_params=pltpu.CompilerParams(needs_layout_passes=False),
        scratch_types=(
            [pltpu.VMEM((V * H,), jnp.float32)]
            + [pltpu.VMEM((rows_per_w,), jnp.int32)]
            + [pltpu.VMEM((CH, H), jnp.float32) for _ in range(_D)]
            + [pltpu.SemaphoreType.DMA for _ in range(2 * _D)]
        ),
    )
    def k(x_hbm, mask_hbm, table_hbm, out_hbm, tab_v, idx_all, *bufs):
        xs = bufs[:_D]
        sem_in = bufs[_D:2 * _D]
        sem_out = bufs[2 * _D:3 * _D]
        wid = lax.axis_index("s") * NC + lax.axis_index("c")
        row0 = wid * rows_per_w
        iota = lax.iota(jnp.int32, _L)

        def in_copy(chunk, k):
            rbase = row0 + chunk * CH
            return pltpu.make_async_copy(
                x_hbm.at[pl.ds(rbase, CH)], xs[k], sem_in[k])

        def start_in(chunk, k):
            in_copy(chunk, k).start()

        def wait_in(chunk, k):
            in_copy(chunk, k).wait()

        def out_copy(chunk, k):
            rbase = row0 + chunk * CH
            return pltpu.make_async_copy(
                xs[k], out_hbm.at[pl.ds(rbase, CH)], sem_out[k])

        def compute(c, k):
            xb = xs[k]
            crow = c * CH

            @plsc.parallel_loop(0, CH)
            def row_body(r):
                mvec = plsc.load_gather(
                    idx_all, [jnp.full((_L,), crow, jnp.int32) + r])
                bvec = mvec * H + iota

                @plsc.parallel_loop(0, groups, unroll=8)
                def col_body(j):
                    t = plsc.load_gather(tab_v, [bvec + j * _L])
                    plsc.addupdate(xb.at[r, pl.ds(j * _L, _L)], t)

        for k in range(_D):
            start_in(k, k)
        pltpu.sync_copy(table_hbm, tab_v)
        pltpu.sync_copy(mask_hbm.at[pl.ds(row0, rows_per_w)], idx_all)

        for k in range(_D):
            wait_in(k, k)
            compute(k, k)
            out_copy(k, k).start()
            if k >= 2:
                pk = k - 2
                out_copy(pk, pk).wait()
                start_in(pk + _D, pk)

        def quad_body(q, carry):
            c0 = q * _D
            for k in range(_D):
                c = c0 + k
                wait_in(c, k)
                compute(c, k)
                out_copy(c, k).start()
                pk = (k - 2) % _D
                out_copy(c - 2, pk).wait()
                start_in(c - 2 + _D, pk)
            return carry

        lax.fori_loop(1, quads - 1, quad_body, 0)

        c0 = (quads - 1) * _D
        for k in range(_D):
            c = c0 + k
            wait_in(c, k)
            compute(c, k)
            out_copy(c, k).start()
            pk = (k - 2) % _D
            out_copy(c - 2, pk).wait()
            if k <= 1:
                start_in(c - 2 + _D, pk)
        out_copy(n_chunks - 2, _D - 2).wait()
        out_copy(n_chunks - 1, _D - 1).wait()

    return k


def kernel(input, segment_mask, segment_embedding):
    B, S, H = input.shape
    V = segment_embedding.shape[0]
    N = B * S
    x = input.reshape(N, H)
    m = segment_mask.reshape(N).astype(jnp.int32)
    tab = segment_embedding.reshape(V * H).astype(jnp.float32)
    out = _make_sc_kernel(N, H, V, 16)(x, m, tab)
    return out.reshape(B, S, H)


# speedup vs baseline: 1.1100x; 1.1100x over previous
import functools

import jax
import jax.numpy as jnp
from jax import lax
from jax.experimental import pallas as pl
from jax.experimental.pallas import tpu as pltpu
from jax.experimental.pallas import tpu_sc as plsc

_L = 16
_DI = 4
_DO = 2


@functools.lru_cache(maxsize=None)
def _make_sc_kernel(N, H, V, CH):
    info = plsc.get_sparse_core_info()
    NC, NS = info.num_cores, info.num_subcores
    NW = NC * NS
    assert N % (NW * CH) == 0 and H % _L == 0
    rows_per_w = N // NW
    n_chunks = rows_per_w // CH
    assert n_chunks % _DI == 0 and n_chunks // _DI >= 3
    quads = n_chunks // _DI
    groups = H // _L
    mesh = plsc.VectorSubcoreMesh(core_axis_name="c", subcore_axis_name="s")

    @functools.partial(
        pl.kernel,
        mesh=mesh,
        out_type=jax.ShapeDtypeStruct((N, H), jnp.float32),
        compiler_params=pltpu.CompilerParams(needs_layout_passes=False),
        scratch_types=(
            [pltpu.VMEM((V * H,), jnp.float32)]
            + [pltpu.VMEM((rows_per_w,), jnp.int32)]
            + [pltpu.VMEM((CH, H), jnp.float32) for _ in range(_DI)]
            + [pltpu.VMEM((CH, H), jnp.float32) for _ in range(_DO)]
            + [pltpu.SemaphoreType.DMA for _ in range(_DI + _DO)]
        ),
    )
    def k(x_hbm, mask_hbm, table_hbm, out_hbm, tab_v, idx_all, *bufs):
        xs = bufs[:_DI]
        os_ = bufs[_DI:_DI + _DO]
        sem_in = bufs[_DI + _DO:2 * _DI + _DO]
        sem_out = bufs[2 * _DI + _DO:]
        wid = lax.axis_index("s") * NC + lax.axis_index("c")
        row0 = wid * rows_per_w
        iota = lax.iota(jnp.int32, _L)

        def in_copy(chunk, ki):
            rbase = row0 + chunk * CH
            return pltpu.make_async_copy(
                x_hbm.at[pl.ds(rbase, CH)], xs[ki], sem_in[ki])

        def out_copy(chunk, ko):
            rbase = row0 + chunk * CH
            return pltpu.make_async_copy(
                os_[ko], out_hbm.at[pl.ds(rbase, CH)], sem_out[ko])

        def compute(c, ki, ko):
            xb, ob = xs[ki], os_[ko]
            crow = c * CH

            @plsc.parallel_loop(0, CH)
            def row_body(r):
                mvec = plsc.load_gather(
                    idx_all, [jnp.full((_L,), crow, jnp.int32) + r])
                bvec = mvec * H + iota

                @plsc.parallel_loop(0, groups, unroll=8)
                def col_body(j):
                    t = plsc.load_gather(tab_v, [bvec + j * _L])
                    sl = pl.ds(j * _L, _L)
                    ob[r, sl] = xb[r, sl] + t

        for k in range(_DI):
            in_copy(k, k).start()
        pltpu.sync_copy(table_hbm, tab_v)
        pltpu.sync_copy(mask_hbm.at[pl.ds(row0, rows_per_w)], idx_all)

        for k in range(_DI):
            in_copy(k, k).wait()
            if k >= _DO:
                out_copy(k - _DO, k % _DO).wait()
            compute(k, k, k % _DO)
            out_copy(k, k % _DO).start()
            in_copy(k + _DI, k).start()

        def quad_body(q, carry):
            c0 = q * _DI
            for k in range(_DI):
                c = c0 + k
                in_copy(c, k).wait()
                out_copy(c - _DO, k % _DO).wait()
                compute(c, k, k % _DO)
                out_copy(c, k % _DO).start()
                in_copy(c + _DI, k).start()
            return carry

        lax.fori_loop(1, quads - 1, quad_body, 0)

        c0 = (quads - 1) * _DI
        for k in range(_DI):
            c = c0 + k
            in_copy(c, k).wait()
            out_copy(c - _DO, k % _DO).wait()
            compute(c, k, k % _DO)
            out_copy(c, k % _DO).start()
        out_copy(n_chunks - 2, 0).wait()
        out_copy(n_chunks - 1, 1).wait()

    return k


def kernel(input, segment_mask, segment_embedding):
    B, S, H = input.shape
    V = segment_embedding.shape[0]
    N = B * S
    x = input.reshape(N, H)
    m = segment_mask.reshape(N).astype(jnp.int32)
    tab = segment_embedding.reshape(V * H).astype(jnp.float32)
    out = _make_sc_kernel(N, H, V, 16)(x, m, tab)
    return out.reshape(B, S, H)
